# Initial kernel scaffold; baseline (speedup 1.0000x reference)
import jax
import jax.numpy as jnp
from jax.experimental import pallas as pl


def kernel(hidden_states, top_k_indices, top_k_gates, q_Wm, q_U, q_S, q_V, q_b, k_Wm, k_U, k_S, k_V, k_b, v_Wm, v_U, v_S, v_V, v_b, o_Wm, o_U, o_S, o_V, o_b):
    raise NotImplementedError("write your pallas kernel here")


# speedup vs baseline: 2.0060x; 2.0060x over previous
import functools

import jax
import jax.numpy as jnp
from jax import lax
from jax.experimental import pallas as pl
from jax.experimental.pallas import tpu as pltpu

B, S, D, H, E, R, K = 4, 577, 1024, 16, 8, 16, 2
DH = D // H
SCALE = DH ** -0.5
SP = 640


def _body(x_ref, xT_ref, wqv_ref, wk_ref, wo_ref, tabRD_ref, tabDR_ref,
          s3_ref, bias_ref, idx_ref, gate_ref, out_ref):
    b = pl.program_id(0)
    f32 = jnp.float32
    bf16 = jnp.bfloat16

    xb = x_ref[0]
    xTb = xT_ref[0]

    e0 = idx_ref[b, 0]
    e1 = idx_ref[b, 1]
    g0 = gate_ref[b, 0]
    g1 = gate_ref[b, 1]

    def lowrank_res(xin_bf, l):
        res = None
        for e, g in ((e0, g0), (e1, g1)):
            i = l * E + e
            dr = tabDR_ref[i].astype(bf16)
            rd = tabRD_ref[i].astype(bf16)
            srow = s3_ref[i] * g
            t = jnp.dot(xin_bf, dr, preferred_element_type=f32)
            t = (t * srow).astype(bf16)
            r = jnp.dot(t, rd, preferred_element_type=f32)
            res = r if res is None else res + r
        return res

    qv = jnp.dot(xb, wqv_ref[...], preferred_element_type=f32)
    kT = jnp.dot(wk_ref[...], xTb, preferred_element_type=f32)

    q = qv[:, :D] + lowrank_res(xb, 0) + bias_ref[0:1, :]
    v = qv[:, D:] + lowrank_res(xb, 2) + bias_ref[1:2, :]

    for e, g in ((e0, g0), (e1, g1)):
        i = E + e
        u = tabDR_ref[i]
        vrow = tabRD_ref[i].astype(bf16)
        us = (u * (s3_ref[i] * g)).astype(bf16)
        tk = jnp.dot(vrow, xTb, preferred_element_type=f32)
        kT = kT + jnp.dot(us, tk.astype(bf16), preferred_element_type=f32)

    qbf = (q * SCALE).astype(bf16)
    vbf = v.astype(bf16)
    kTbf = kT.astype(bf16)

    col = lax.broadcasted_iota(jnp.int32, (SP, SP), 1)
    kmask = col < S
    ctx_parts = []
    for h in range(H):
        qh = qbf[:, h * DH:(h + 1) * DH]
        kTh = kTbf[h * DH:(h + 1) * DH, :]
        s = jnp.dot(qh, kTh, preferred_element_type=f32)
        s = jnp.where(kmask, s, -1e30)
        m = jnp.max(s, axis=1, keepdims=True)
        p = jnp.exp(s - m)
        den = jnp.sum(p, axis=1, keepdims=True)
        vh = vbf[:, h * DH:(h + 1) * DH]
        c = jnp.dot(p.astype(bf16), vh, preferred_element_type=f32)
        ctx_parts.append(c / den)
    ctx = jnp.concatenate(ctx_parts, axis=1)
    ctxbf = ctx.astype(bf16)

    out = jnp.dot(ctxbf, wo_ref[...], preferred_element_type=f32)
    out = out + lowrank_res(ctxbf, 3) + bias_ref[2:3, :]
    out_ref[0] = out


@functools.partial(jax.jit, static_argnums=())
def kernel(hidden_states, top_k_indices, top_k_gates,
           q_Wm, q_U, q_S, q_V, q_b,
           k_Wm, k_U, k_S, k_V, k_b,
           v_Wm, v_U, v_S, v_V, v_b,
           o_Wm, o_U, o_S, o_V, o_b):
    bf16 = jnp.bfloat16
    f32 = jnp.float32

    x = jnp.pad(hidden_states, ((0, 0), (0, SP - S), (0, 0)))
    xbf = x.astype(bf16)
    xT = jnp.transpose(xbf, (0, 2, 1))

    wqv = jnp.concatenate([q_Wm.T, v_Wm.T], axis=1).astype(bf16)
    wk = k_Wm.astype(bf16)
    wo = o_Wm.T.astype(bf16)

    tabRD = jnp.concatenate([
        jnp.transpose(q_U, (0, 2, 1)), k_V,
        jnp.transpose(v_U, (0, 2, 1)), jnp.transpose(o_U, (0, 2, 1)),
    ], axis=0).astype(f32)
    tabDR = jnp.concatenate([
        jnp.transpose(q_V, (0, 2, 1)), k_U,
        jnp.transpose(v_V, (0, 2, 1)), jnp.transpose(o_V, (0, 2, 1)),
    ], axis=0).astype(f32)
    s3 = jnp.concatenate([q_S, k_S, v_S, o_S], axis=0).reshape(4 * E, 1, R)
    bias = jnp.stack([q_b, v_b, o_b], axis=0)

    grid = (B,)
    out = pl.pallas_call(
        _body,
        grid=grid,
        in_specs=[
            pl.BlockSpec((1, SP, D), lambda b: (b, 0, 0)),
            pl.BlockSpec((1, D, SP), lambda b: (b, 0, 0)),
            pl.BlockSpec((D, 2 * D), lambda b: (0, 0)),
            pl.BlockSpec((D, D), lambda b: (0, 0)),
            pl.BlockSpec((D, D), lambda b: (0, 0)),
            pl.BlockSpec((4 * E, R, D), lambda b: (0, 0, 0)),
            pl.BlockSpec((4 * E, D, R), lambda b: (0, 0, 0)),
            pl.BlockSpec((4 * E, 1, R), lambda b: (0, 0, 0)),
            pl.BlockSpec((3, D), lambda b: (0, 0)),
            pl.BlockSpec(memory_space=pltpu.SMEM),
            pl.BlockSpec(memory_space=pltpu.SMEM),
        ],
        out_specs=pl.BlockSpec((1, SP, D), lambda b: (b, 0, 0)),
        out_shape=jax.ShapeDtypeStruct((B, SP, D), f32),
        compiler_params=pltpu.CompilerParams(
            dimension_semantics=("arbitrary",),
            vmem_limit_bytes=100 * 1024 * 1024,
        ),
    )(xbf, xT, wqv, wk, wo, tabRD, tabDR, s3, bias,
      top_k_indices.astype(jnp.int32), top_k_gates)
    return out[:, :S, :]


# speedup vs baseline: 2.3167x; 1.1549x over previous
import functools

import jax
import jax.numpy as jnp
from jax import lax
from jax.experimental import pallas as pl
from jax.experimental.pallas import tpu as pltpu

B, S, D, H, E, R, K = 4, 577, 1024, 16, 8, 16, 2
DH = D // H
SCALE = DH ** -0.5
SP = 640

_TRANS_RHS = (((1,), (1,)), ((), ()))


def _body(x_ref, wqkv_ref, wo_ref, tabU_ref, tabV_ref,
          s3_ref, bias_ref, idx_ref, gate_ref, out_ref):
    b = pl.program_id(0)
    f32 = jnp.float32
    bf16 = jnp.bfloat16

    row = lax.broadcasted_iota(jnp.int32, (SP, 1), 0)
    xb = jnp.where(row < S, x_ref[0], 0.0).astype(bf16)

    e0 = idx_ref[b, 0]
    e1 = idx_ref[b, 1]
    g0 = gate_ref[b, 0]
    g1 = gate_ref[b, 1]

    def lowrank_res(xin_bf, l):
        res = None
        for e, g in ((e0, g0), (e1, g1)):
            i = l * E + e
            vslab = tabV_ref[i]
            uslab = tabU_ref[i]
            srow = s3_ref[i] * g
            t = lax.dot_general(xin_bf, vslab, _TRANS_RHS,
                                preferred_element_type=f32)
            t = (t * srow).astype(bf16)
            r = lax.dot_general(t, uslab, _TRANS_RHS,
                                preferred_element_type=f32)
            res = r if res is None else res + r
        return res

    qkv = lax.dot_general(xb, wqkv_ref[...], _TRANS_RHS,
                          preferred_element_type=f32)

    q = qkv[:, :D] + lowrank_res(xb, 0) + bias_ref[0:1, :]
    k = qkv[:, D:2 * D] + lowrank_res(xb, 1)
    v = qkv[:, 2 * D:] + lowrank_res(xb, 2) + bias_ref[1:2, :]

    qbf = (q * SCALE).astype(bf16)
    kbf = k.astype(bf16)
    vbf = v.astype(bf16)

    col = lax.broadcasted_iota(jnp.int32, (SP, SP), 1)
    kmask = col < S
    ctx_parts = []
    for h in range(H):
        qh = qbf[:, h * DH:(h + 1) * DH]
        kh = kbf[:, h * DH:(h + 1) * DH]
        s = lax.dot_general(qh, kh, _TRANS_RHS,
                            preferred_element_type=f32)
        s = jnp.where(kmask, s, -1e30)
        m = jnp.max(s, axis=1, keepdims=True)
        p = jnp.exp(s - m)
        den = jnp.sum(p, axis=1, keepdims=True)
        vh = vbf[:, h * DH:(h + 1) * DH]
        c = jnp.dot(p.astype(bf16), vh, preferred_element_type=f32)
        ctx_parts.append(c / den)
    ctx = jnp.concatenate(ctx_parts, axis=1)
    ctxbf = ctx.astype(bf16)

    out = lax.dot_general(ctxbf, wo_ref[...], _TRANS_RHS,
                          preferred_element_type=f32)
    out = out + lowrank_res(ctxbf, 3) + bias_ref[2:3, :]
    out_ref[0] = out[:S, :]


@functools.partial(jax.jit, static_argnums=())
def kernel(hidden_states, top_k_indices, top_k_gates,
           q_Wm, q_U, q_S, q_V, q_b,
           k_Wm, k_U, k_S, k_V, k_b,
           v_Wm, v_U, v_S, v_V, v_b,
           o_Wm, o_U, o_S, o_V, o_b):
    bf16 = jnp.bfloat16
    f32 = jnp.float32

    wqkv = jnp.concatenate([q_Wm, k_Wm, v_Wm], axis=0).astype(bf16)
    wo = o_Wm.astype(bf16)

    tabU = jnp.concatenate([q_U, k_U, v_U, o_U], axis=0).astype(bf16)
    tabV = jnp.concatenate([q_V, k_V, v_V, o_V], axis=0).astype(bf16)
    s3 = jnp.concatenate([q_S, k_S, v_S, o_S], axis=0).reshape(4 * E, 1, R)
    bias = jnp.stack([q_b, v_b, o_b], axis=0)

    out = pl.pallas_call(
        _body,
        grid=(B,),
        in_specs=[
            pl.BlockSpec((1, SP, D), lambda b: (b, 0, 0)),
            pl.BlockSpec((3 * D, D), lambda b: (0, 0)),
            pl.BlockSpec((D, D), lambda b: (0, 0)),
            pl.BlockSpec((4 * E, D, R), lambda b: (0, 0, 0)),
            pl.BlockSpec((4 * E, R, D), lambda b: (0, 0, 0)),
            pl.BlockSpec((4 * E, 1, R), lambda b: (0, 0, 0)),
            pl.BlockSpec((3, D), lambda b: (0, 0)),
            pl.BlockSpec(memory_space=pltpu.SMEM),
            pl.BlockSpec(memory_space=pltpu.SMEM),
        ],
        out_specs=pl.BlockSpec((1, S, D), lambda b: (b, 0, 0)),
        out_shape=jax.ShapeDtypeStruct((B, S, D), f32),
        compiler_params=pltpu.CompilerParams(
            dimension_semantics=("arbitrary",),
            vmem_limit_bytes=100 * 1024 * 1024,
        ),
    )(hidden_states, wqkv, wo, tabU, tabV, s3, bias,
      top_k_indices.astype(jnp.int32), top_k_gates)
    return out


# speedup vs baseline: 2.3865x; 1.0301x over previous
import functools

import jax
import jax.numpy as jnp
from jax import lax
from jax.experimental import pallas as pl
from jax.experimental.pallas import tpu as pltpu

B, S, D, H, E, R, K = 4, 577, 1024, 16, 8, 16, 2
DH = D // H
SCALE = DH ** -0.5
SP = 640

_TRANS_RHS = (((1,), (1,)), ((), ()))


def _prep_body(qW_ref, kW_ref, vW_ref, oW_ref,
               qU_ref, kU_ref, vU_ref, oU_ref,
               qV_ref, kV_ref, vV_ref, oV_ref,
               wqkv_ref, wo_ref, tabU_ref, tabV_ref):
    bf16 = jnp.bfloat16
    wqkv_ref[:D] = qW_ref[...].astype(bf16)
    wqkv_ref[D:2 * D] = kW_ref[...].astype(bf16)
    wqkv_ref[2 * D:] = vW_ref[...].astype(bf16)
    wo_ref[...] = oW_ref[...].astype(bf16)
    for i, r in enumerate((qU_ref, kU_ref, vU_ref, oU_ref)):
        tabU_ref[i * E:(i + 1) * E] = r[...].astype(bf16)
    for i, r in enumerate((qV_ref, kV_ref, vV_ref, oV_ref)):
        tabV_ref[i * E:(i + 1) * E] = r[...].astype(bf16)


def _body(x_ref, wqkv_ref, wo_ref, tabU_ref, tabV_ref,
          qS_ref, kS_ref, vS_ref, oS_ref,
          qb_ref, vb_ref, ob_ref, idx_ref, gate_ref, out_ref):
    b = pl.program_id(0)
    f32 = jnp.float32
    bf16 = jnp.bfloat16

    row = lax.broadcasted_iota(jnp.int32, (SP, 1), 0)
    xb = jnp.where(row < S, x_ref[0], 0.0).astype(bf16)

    e0 = idx_ref[b, 0]
    e1 = idx_ref[b, 1]
    g0 = gate_ref[b, 0]
    g1 = gate_ref[b, 1]

    def lowrank_res(xin_bf, l, s_ref):
        res = None
        for e, g in ((e0, g0), (e1, g1)):
            i = l * E + e
            vslab = tabV_ref[i]
            uslab = tabU_ref[i]
            srow = (s_ref[e] * g).reshape(1, R)
            t = lax.dot_general(xin_bf, vslab, _TRANS_RHS,
                                preferred_element_type=f32)
            t = (t * srow).astype(bf16)
            r = lax.dot_general(t, uslab, _TRANS_RHS,
                                preferred_element_type=f32)
            res = r if res is None else res + r
        return res

    qkv = lax.dot_general(xb, wqkv_ref[...], _TRANS_RHS,
                          preferred_element_type=f32)

    q = qkv[:, :D] + lowrank_res(xb, 0, qS_ref) + qb_ref[...].reshape(1, D)
    k = qkv[:, D:2 * D] + lowrank_res(xb, 1, kS_ref)
    v = qkv[:, 2 * D:] + lowrank_res(xb, 2, vS_ref) + vb_ref[...].reshape(1, D)

    qbf = (q * SCALE).astype(bf16)
    kbf = k.astype(bf16)
    vbf = v.astype(bf16)

    col = lax.broadcasted_iota(jnp.int32, (SP, SP), 1)
    kmask = col < S
    ctx_parts = []
    for h in range(H):
        qh = qbf[:, h * DH:(h + 1) * DH]
        kh = kbf[:, h * DH:(h + 1) * DH]
        s = lax.dot_general(qh, kh, _TRANS_RHS,
                            preferred_element_type=f32)
        s = jnp.where(kmask, s, -1e30)
        m = jnp.max(s, axis=1, keepdims=True)
        p = jnp.exp(s - m)
        den = jnp.sum(p, axis=1, keepdims=True)
        vh = vbf[:, h * DH:(h + 1) * DH]
        c = jnp.dot(p.astype(bf16), vh, preferred_element_type=f32)
        ctx_parts.append(c / den)
    ctx = jnp.concatenate(ctx_parts, axis=1)
    ctxbf = ctx.astype(bf16)

    out = lax.dot_general(ctxbf, wo_ref[...], _TRANS_RHS,
                          preferred_element_type=f32)
    out = out + lowrank_res(ctxbf, 3, oS_ref) + ob_ref[...].reshape(1, D)
    out_ref[0] = out[:S, :]


@functools.partial(jax.jit, static_argnums=())
def kernel(hidden_states, top_k_indices, top_k_gates,
           q_Wm, q_U, q_S, q_V, q_b,
           k_Wm, k_U, k_S, k_V, k_b,
           v_Wm, v_U, v_S, v_V, v_b,
           o_Wm, o_U, o_S, o_V, o_b):
    bf16 = jnp.bfloat16
    f32 = jnp.float32

    wqkv, wo, tabU, tabV = pl.pallas_call(
        _prep_body,
        out_shape=[
            jax.ShapeDtypeStruct((3 * D, D), bf16),
            jax.ShapeDtypeStruct((D, D), bf16),
            jax.ShapeDtypeStruct((4 * E, D, R), bf16),
            jax.ShapeDtypeStruct((4 * E, R, D), bf16),
        ],
        compiler_params=pltpu.CompilerParams(
            vmem_limit_bytes=100 * 1024 * 1024,
        ),
    )(q_Wm, k_Wm, v_Wm, o_Wm, q_U, k_U, v_U, o_U, q_V, k_V, v_V, o_V)

    out = pl.pallas_call(
        _body,
        grid=(B,),
        in_specs=[
            pl.BlockSpec((1, SP, D), lambda b: (b, 0, 0)),
            pl.BlockSpec((3 * D, D), lambda b: (0, 0)),
            pl.BlockSpec((D, D), lambda b: (0, 0)),
            pl.BlockSpec((4 * E, D, R), lambda b: (0, 0, 0)),
            pl.BlockSpec((4 * E, R, D), lambda b: (0, 0, 0)),
            pl.BlockSpec((E, R), lambda b: (0, 0)),
            pl.BlockSpec((E, R), lambda b: (0, 0)),
            pl.BlockSpec((E, R), lambda b: (0, 0)),
            pl.BlockSpec((E, R), lambda b: (0, 0)),
            pl.BlockSpec((D,), lambda b: (0,)),
            pl.BlockSpec((D,), lambda b: (0,)),
            pl.BlockSpec((D,), lambda b: (0,)),
            pl.BlockSpec(memory_space=pltpu.SMEM),
            pl.BlockSpec(memory_space=pltpu.SMEM),
        ],
        out_specs=pl.BlockSpec((1, S, D), lambda b: (b, 0, 0)),
        out_shape=jax.ShapeDtypeStruct((B, S, D), f32),
        compiler_params=pltpu.CompilerParams(
            dimension_semantics=("arbitrary",),
            vmem_limit_bytes=100 * 1024 * 1024,
        ),
    )(hidden_states, wqkv, wo, tabU, tabV,
      q_S, k_S, v_S, o_S, q_b, v_b, o_b,
      top_k_indices, top_k_gates)
    return out
